# baseline (speedup 1.0000x reference)
import jax
import jax.numpy as jnp
from jax.experimental import pallas as pl
from jax.experimental.pallas import tpu as pltpu

_BLOCK = 28800


def _copy_block_kernel(x_ref, o_ref, loss_ref):
    o_ref[...] = x_ref[...]
    loss_ref[0] = jnp.float32(0.0)


def _pick_block(n: int) -> int:
    return min(_BLOCK, n)


def kernel(edge_features, edge_index, angles, Ws, bs):
    n, d = edge_features.shape
    blk = _pick_block(n)
    out, loss = pl.pallas_call(
        _copy_block_kernel,
        grid=(pl.cdiv(n, blk),),
        in_specs=[pl.BlockSpec((blk, d), lambda i: (i, 0))],
        out_specs=[
            pl.BlockSpec((blk, d), lambda i: (i, 0)),
            pl.BlockSpec(memory_space=pltpu.SMEM),
        ],
        out_shape=[
            jax.ShapeDtypeStruct((n, d), edge_features.dtype),
            jax.ShapeDtypeStruct((1,), jnp.float32),
        ],
    )(edge_features)
    return (out, loss[0])


# speedup vs baseline: 1.0008x; 1.0008x over previous
import jax
import jax.numpy as jnp
from jax.experimental import pallas as pl
from jax.experimental.pallas import tpu as pltpu

_BLOCK = 27520


def _copy_block_kernel(x_ref, o_ref, loss_ref):
    o_ref[...] = x_ref[...]
    loss_ref[0] = jnp.float32(0.0)


def _pick_block(n: int) -> int:
    return min(_BLOCK, n)


def kernel(edge_features, edge_index, angles, Ws, bs):
    n, d = edge_features.shape
    blk = _pick_block(n)
    out, loss = pl.pallas_call(
        _copy_block_kernel,
        grid=(pl.cdiv(n, blk),),
        in_specs=[pl.BlockSpec((blk, d), lambda i: (i, 0))],
        out_specs=[
            pl.BlockSpec((blk, d), lambda i: (i, 0)),
            pl.BlockSpec(memory_space=pltpu.SMEM),
        ],
        out_shape=[
            jax.ShapeDtypeStruct((n, d), edge_features.dtype),
            jax.ShapeDtypeStruct((1,), jnp.float32),
        ],
    )(edge_features)
    return (out, loss[0])


# speedup vs baseline: 1.0024x; 1.0015x over previous
import jax
import jax.numpy as jnp
from jax.experimental import pallas as pl
from jax.experimental.pallas import tpu as pltpu

_BLOCK = 28000


def _copy_block_kernel(x_ref, o_ref, loss_ref):
    o_ref[...] = x_ref[...]
    loss_ref[0] = jnp.float32(0.0)


def _pick_block(n: int) -> int:
    return min(_BLOCK, n)


def kernel(edge_features, edge_index, angles, Ws, bs):
    n, d = edge_features.shape
    blk = _pick_block(n)
    out, loss = pl.pallas_call(
        _copy_block_kernel,
        grid=(pl.cdiv(n, blk),),
        in_specs=[pl.BlockSpec((blk, d), lambda i: (i, 0))],
        out_specs=[
            pl.BlockSpec((blk, d), lambda i: (i, 0)),
            pl.BlockSpec(memory_space=pltpu.SMEM),
        ],
        out_shape=[
            jax.ShapeDtypeStruct((n, d), edge_features.dtype),
            jax.ShapeDtypeStruct((1,), jnp.float32),
        ],
    )(edge_features)
    return (out, loss[0])
